# baseline (speedup 1.0000x reference)
import functools

import jax
import jax.numpy as jnp
from jax import lax
from jax.experimental import pallas as pl
from jax.experimental.pallas import tpu as pltpu
from jax.experimental.pallas import tpu_sc as plsc

R = 64
N = 8192
K = 256
L = 16
NV = N // L
NC = 2
NS = 16
ROWS_PER_W = R // (NC * NS)


def _sc_body(x_hbm, o_hbm, x_v0, x_v1, key_v, cand0_v, cand1_v, hist_v,
             totals_v, out_v0, out_v1, sem_in0, sem_in1, sem_out0, sem_out1):
    MIN32 = jnp.int32(-2147483648)
    lane = jnp.arange(L, dtype=jnp.int32)
    ones = jnp.ones((L,), jnp.int32)
    zeros = jnp.zeros((L,), jnp.int32)
    zidx = jnp.zeros((L,), jnp.int32)
    kk = jnp.int32(K)

    wid = lax.axis_index("s") * NC + lax.axis_index("c")

    def splat0(v):
        return v.at[zidx].get(mode="promise_in_bounds")

    def select_bucket(k_rem):
        @plsc.parallel_loop(0, 16)
        def _tot(j):
            acc = hist_v[pl.ds(j * L, L)]
            hist_v[pl.ds(j * L, L)] = zeros
            for l in range(1, L):
                acc = acc + hist_v[pl.ds(l * 256 + j * L, L)]
                hist_v[pl.ds(l * 256 + j * L, L)] = zeros
            totals_v[pl.ds(j * L, L)] = acc
        above = zeros
        pcacc = zeros
        sufsel = zeros
        totsel = zeros
        b_hi = zeros
        taken = zeros
        for j in range(15, -1, -1):
            tj = totals_v[pl.ds(j * L, L)]
            rc = lax.rev(plsc.cumsum(lax.rev(tj, (0,))), (0,))
            suf = rc + above
            hit = suf >= k_rem
            hit_i = hit.astype(jnp.int32)
            pcacc = pcacc + hit_i
            above = above + splat0(rc)
            anyhit = splat0(hit_i)
            upd = (anyhit * (jnp.int32(1) - taken)) > 0
            sufsel = jnp.where(upd & hit, suf, sufsel)
            totsel = jnp.where(upd & hit, tj, totsel)
            b_hi = jnp.where(upd & hit, lane + jnp.int32(j * L), b_hi)
            taken = jnp.where(upd, ones, taken)
        b_star = jnp.sum(pcacc) - jnp.int32(1)
        sel = (b_hi == b_star).astype(jnp.int32)
        tot_b = jnp.max(sel * totsel)
        suf_b = jnp.max(sel * sufsel)
        c_above = suf_b - tot_b
        return b_star, k_rem - c_above

    def compute_row(x_v, out_v):
        @plsc.parallel_loop(0, NV, unroll=8)
        def p1(i):
            xv = x_v[pl.ds(i * L, L)]
            b = lax.bitcast_convert_type(xv, jnp.int32)
            sk = jnp.where(b >= 0, b, MIN32 - b)
            key_v[pl.ds(i * L, L)] = sk
            bucket = (sk >> 24) + jnp.int32(128)
            plsc.addupdate_scatter(hist_v, [lane * 256 + bucket], ones)
        b0, k2 = select_bucket(kk)

        @plsc.parallel_loop(0, NV, unroll=8, carry=zeros)
        def p2(i, off):
            sk = key_v[pl.ds(i * L, L)]
            m = ((sk >> 24) + jnp.int32(128)) == b0
            plsc.store_scatter(cand0_v, [lane * NV + off], sk, mask=m)
            b1 = (sk >> 16) & jnp.int32(0xFF)
            plsc.addupdate_scatter(hist_v, [lane * 256 + b1], ones, mask=m)
            return off + m.astype(jnp.int32)
        off0 = p2
        b1s, k3 = select_bucket(k2)

        max0 = jnp.max(off0)

        @plsc.parallel_loop(0, max0, carry=zeros)
        def p3(i, off):
            valid = i < off0
            sk = plsc.load_gather(cand0_v, [lane * NV + i], mask=valid)
            m = valid & (((sk >> 16) & jnp.int32(0xFF)) == b1s)
            plsc.store_scatter(cand1_v, [lane * NV + off], sk, mask=m)
            b2 = (sk >> 8) & jnp.int32(0xFF)
            plsc.addupdate_scatter(hist_v, [lane * 256 + b2], ones, mask=m)
            return off + m.astype(jnp.int32)
        off1 = p3
        b2s, k4 = select_bucket(k3)

        max1 = jnp.max(off1)

        @plsc.parallel_loop(0, max1)
        def p4(i):
            valid = i < off1
            sk = plsc.load_gather(cand1_v, [lane * NV + i], mask=valid)
            m = valid & (((sk >> 8) & jnp.int32(0xFF)) == b2s)
            b3 = sk & jnp.int32(0xFF)
            plsc.addupdate_scatter(hist_v, [lane * 256 + b3], ones, mask=m)
        b3s, need = select_bucket(k4)

        t = (lax.shift_left(b0 - jnp.int32(128), jnp.int32(24))
             | lax.shift_left(b1s, jnp.int32(16))
             | lax.shift_left(b2s, jnp.int32(8)) | b3s)

        @plsc.parallel_loop(0, NV, unroll=8, carry=zeros)
        def pout(i, cnt):
            sk = key_v[pl.ds(i * L, L)]
            xv = x_v[pl.ds(i * L, L)]
            ge = sk >= t
            out_v[pl.ds(i * L, L)] = jnp.where(ge, xv, jnp.float32(0.0))
            return cnt + ge.astype(jnp.int32)
        total_ge = jnp.sum(pout)

        @pl.when(total_ge != kk)
        def _fixup():
            def pfix(i, c):
                sk = key_v[pl.ds(i * L, L)]
                xv = x_v[pl.ds(i * L, L)]
                eq = sk == t
                eq_i = eq.astype(jnp.int32)
                pre = plsc.cumsum(eq_i) + c
                keep = (sk > t) | (eq & (pre <= need))
                out_v[pl.ds(i * L, L)] = jnp.where(keep, xv, jnp.float32(0.0))
                return c + jnp.sum(eq_i)
            lax.fori_loop(0, NV, pfix, jnp.int32(0))

    @plsc.parallel_loop(0, 256, unroll=8)
    def _clr(i):
        hist_v[pl.ds(i * L, L)] = zeros

    row0 = wid * ROWS_PER_W
    row1 = row0 + 1
    in0 = pltpu.async_copy(x_hbm.at[row0], x_v0, sem_in0)
    in1 = pltpu.async_copy(x_hbm.at[row1], x_v1, sem_in1)
    in0.wait()
    compute_row(x_v0, out_v0)
    w0 = pltpu.async_copy(out_v0, o_hbm.at[row0], sem_out0)
    in1.wait()
    compute_row(x_v1, out_v1)
    w1 = pltpu.async_copy(out_v1, o_hbm.at[row1], sem_out1)
    w0.wait()
    w1.wait()


def kernel(x, sparse_dim):
    del sparse_dim
    mesh = plsc.VectorSubcoreMesh(core_axis_name="c", subcore_axis_name="s",
                                  num_cores=NC, num_subcores=NS)
    f = pl.kernel(
        _sc_body,
        out_type=jax.ShapeDtypeStruct((R, N), jnp.float32),
        mesh=mesh,
        scratch_types=[
            pltpu.VMEM((N,), jnp.float32),
            pltpu.VMEM((N,), jnp.float32),
            pltpu.VMEM((N,), jnp.int32),
            pltpu.VMEM((N,), jnp.int32),
            pltpu.VMEM((N,), jnp.int32),
            pltpu.VMEM((L * 256,), jnp.int32),
            pltpu.VMEM((256,), jnp.int32),
            pltpu.VMEM((N,), jnp.float32),
            pltpu.VMEM((N,), jnp.float32),
            pltpu.SemaphoreType.DMA,
            pltpu.SemaphoreType.DMA,
            pltpu.SemaphoreType.DMA,
            pltpu.SemaphoreType.DMA,
        ],
        compiler_params=pltpu.CompilerParams(use_tc_tiling_on_sc=False,
                                             needs_layout_passes=False),
    )
    return f(x)


# speedup vs baseline: 1.1912x; 1.1912x over previous
import functools

import jax
import jax.numpy as jnp
from jax import lax
from jax.experimental import pallas as pl
from jax.experimental.pallas import tpu as pltpu
from jax.experimental.pallas import tpu_sc as plsc

R = 64
N = 8192
K = 256
L = 16
NV = N // L
NC = 2
NS = 16
R_SC = NC * NS
R_TC = R - R_SC


def _sc_body(x_hbm, o_hbm, x_v, key_v, cand0_v, cand1_v, hist_v,
             totals_v, out_v, sem_in, sem_out):
    MIN32 = jnp.int32(-2147483648)
    lane = jnp.arange(L, dtype=jnp.int32)
    ones = jnp.ones((L,), jnp.int32)
    zeros = jnp.zeros((L,), jnp.int32)
    zidx = jnp.zeros((L,), jnp.int32)
    kk = jnp.int32(K)

    wid = lax.axis_index("s") * NC + lax.axis_index("c")

    def splat0(v):
        return v.at[zidx].get(mode="promise_in_bounds")

    def select_bucket(k_rem):
        @plsc.parallel_loop(0, 16)
        def _tot(j):
            acc = hist_v[pl.ds(j * L, L)]
            hist_v[pl.ds(j * L, L)] = zeros
            for l in range(1, L):
                acc = acc + hist_v[pl.ds(l * 256 + j * L, L)]
                hist_v[pl.ds(l * 256 + j * L, L)] = zeros
            totals_v[pl.ds(j * L, L)] = acc
        above = zeros
        pcacc = zeros
        sufsel = zeros
        totsel = zeros
        b_hi = zeros
        taken = zeros
        for j in range(15, -1, -1):
            tj = totals_v[pl.ds(j * L, L)]
            rc = lax.rev(plsc.cumsum(lax.rev(tj, (0,))), (0,))
            suf = rc + above
            hit = suf >= k_rem
            hit_i = hit.astype(jnp.int32)
            pcacc = pcacc + hit_i
            above = above + splat0(rc)
            anyhit = splat0(hit_i)
            upd = (anyhit * (jnp.int32(1) - taken)) > 0
            sufsel = jnp.where(upd & hit, suf, sufsel)
            totsel = jnp.where(upd & hit, tj, totsel)
            b_hi = jnp.where(upd & hit, lane + jnp.int32(j * L), b_hi)
            taken = jnp.where(upd, ones, taken)
        b_star = jnp.sum(pcacc) - jnp.int32(1)
        sel = (b_hi == b_star).astype(jnp.int32)
        tot_b = jnp.max(sel * totsel)
        suf_b = jnp.max(sel * sufsel)
        c_above = suf_b - tot_b
        return b_star, k_rem - c_above

    @plsc.parallel_loop(0, 256, unroll=8)
    def _clr(i):
        hist_v[pl.ds(i * L, L)] = zeros

    pltpu.async_copy(x_hbm.at[wid], x_v, sem_in).wait()

    @plsc.parallel_loop(0, NV, unroll=8)
    def p1(i):
        xv = x_v[pl.ds(i * L, L)]
        b = lax.bitcast_convert_type(xv, jnp.int32)
        sk = jnp.where(b >= 0, b, MIN32 - b)
        key_v[pl.ds(i * L, L)] = sk
        bucket = (sk >> 24) + jnp.int32(128)
        plsc.addupdate_scatter(hist_v, [lane * 256 + bucket], ones)
    b0, k2 = select_bucket(kk)

    @plsc.parallel_loop(0, NV, unroll=8, carry=zeros)
    def p2(i, off):
        sk = key_v[pl.ds(i * L, L)]
        m = ((sk >> 24) + jnp.int32(128)) == b0
        plsc.store_scatter(cand0_v, [lane * NV + off], sk, mask=m)
        b1 = (sk >> 16) & jnp.int32(0xFF)
        plsc.addupdate_scatter(hist_v, [lane * 256 + b1], ones, mask=m)
        return off + m.astype(jnp.int32)
    off0 = p2
    b1s, k3 = select_bucket(k2)

    max0 = jnp.max(off0)

    @plsc.parallel_loop(0, max0, carry=zeros)
    def p3(i, off):
        valid = i < off0
        sk = plsc.load_gather(cand0_v, [lane * NV + i], mask=valid)
        m = valid & (((sk >> 16) & jnp.int32(0xFF)) == b1s)
        plsc.store_scatter(cand1_v, [lane * NV + off], sk, mask=m)
        b2 = (sk >> 8) & jnp.int32(0xFF)
        plsc.addupdate_scatter(hist_v, [lane * 256 + b2], ones, mask=m)
        return off + m.astype(jnp.int32)
    off1 = p3
    b2s, k4 = select_bucket(k3)

    max1 = jnp.max(off1)

    @plsc.parallel_loop(0, max1)
    def p4(i):
        valid = i < off1
        sk = plsc.load_gather(cand1_v, [lane * NV + i], mask=valid)
        m = valid & (((sk >> 8) & jnp.int32(0xFF)) == b2s)
        b3 = sk & jnp.int32(0xFF)
        plsc.addupdate_scatter(hist_v, [lane * 256 + b3], ones, mask=m)
    b3s, need = select_bucket(k4)

    t = (lax.shift_left(b0 - jnp.int32(128), jnp.int32(24))
         | lax.shift_left(b1s, jnp.int32(16))
         | lax.shift_left(b2s, jnp.int32(8)) | b3s)

    @plsc.parallel_loop(0, NV, unroll=8, carry=zeros)
    def pout(i, cnt):
        sk = key_v[pl.ds(i * L, L)]
        xv = x_v[pl.ds(i * L, L)]
        ge = sk >= t
        out_v[pl.ds(i * L, L)] = jnp.where(ge, xv, jnp.float32(0.0))
        return cnt + ge.astype(jnp.int32)
    total_ge = jnp.sum(pout)

    @pl.when(total_ge != kk)
    def _fixup():
        def pfix(i, c):
            sk = key_v[pl.ds(i * L, L)]
            xv = x_v[pl.ds(i * L, L)]
            eq = sk == t
            eq_i = eq.astype(jnp.int32)
            pre = plsc.cumsum(eq_i) + c
            keep = (sk > t) | (eq & (pre <= need))
            out_v[pl.ds(i * L, L)] = jnp.where(keep, xv, jnp.float32(0.0))
            return c + jnp.sum(eq_i)
        lax.fori_loop(0, NV, pfix, jnp.int32(0))

    pltpu.async_copy(out_v, o_hbm.at[wid], sem_out).wait()


def _sc_call(x_sc):
    mesh = plsc.VectorSubcoreMesh(core_axis_name="c", subcore_axis_name="s",
                                  num_cores=NC, num_subcores=NS)
    f = pl.kernel(
        _sc_body,
        out_type=jax.ShapeDtypeStruct((R_SC, N), jnp.float32),
        mesh=mesh,
        scratch_types=[
            pltpu.VMEM((N,), jnp.float32),
            pltpu.VMEM((N,), jnp.int32),
            pltpu.VMEM((N,), jnp.int32),
            pltpu.VMEM((N,), jnp.int32),
            pltpu.VMEM((L * 256,), jnp.int32),
            pltpu.VMEM((256,), jnp.int32),
            pltpu.VMEM((N,), jnp.float32),
            pltpu.SemaphoreType.DMA,
            pltpu.SemaphoreType.DMA,
        ],
        compiler_params=pltpu.CompilerParams(use_tc_tiling_on_sc=False,
                                             needs_layout_passes=False),
    )
    return f(x_sc)


def _tc_body(x_ref, o_ref):
    MIN32 = jnp.int32(-2147483648)
    x = x_ref[...]
    n_rows, n_cols = x.shape
    bits = lax.bitcast_convert_type(x, jnp.int32)
    skey = jnp.where(bits >= 0, bits, MIN32 - bits)

    k = jnp.int32(K)

    def vstep(i, u):
        bit = jnp.int32(31) - i
        cand = u | (jnp.int32(1) << bit)
        t = cand ^ MIN32
        cnt = jnp.sum((skey >= t).astype(jnp.int32), axis=1, keepdims=True)
        return jnp.where(cnt >= k, cand, u)

    u0 = jnp.zeros((n_rows, 1), jnp.int32)
    u = lax.fori_loop(0, 32, vstep, u0)
    t = u ^ MIN32

    gt = skey > t
    eq = skey == t
    needed = k - jnp.sum(gt.astype(jnp.int32), axis=1, keepdims=True)

    col = lax.broadcasted_iota(jnp.int32, (n_rows, n_cols), 1)
    eq_i = eq.astype(jnp.int32)

    def istep(i, m):
        bit = jnp.int32(13) - i
        cand = m | (jnp.int32(1) << bit)
        cnt = jnp.sum(jnp.where(col < cand, eq_i, 0), axis=1, keepdims=True)
        return jnp.where(cnt <= needed, cand, m)

    m = lax.fori_loop(0, 14, istep, jnp.zeros((n_rows, 1), jnp.int32))

    keep = gt | (eq & (col < m))
    o_ref[...] = jnp.where(keep, x, jnp.float32(0.0))


def _tc_call(x_tc):
    return pl.pallas_call(
        _tc_body,
        out_shape=jax.ShapeDtypeStruct((R_TC, N), jnp.float32),
    )(x_tc)


def kernel(x, sparse_dim):
    del sparse_dim
    out_sc = _sc_call(x[:R_SC])
    out_tc = _tc_call(x[R_SC:])
    return jnp.concatenate([out_sc, out_tc], axis=0)
